# baseline (speedup 1.0000x reference)
import functools

import jax
import jax.numpy as jnp
from jax import lax
from jax.experimental import pallas as pl
from jax.experimental.pallas import tpu as pltpu
from jax.experimental.pallas import tpu_sc as plsc

N = 16384
C = 1000
NW = 32
RPW = N // NW
CH = 32
NCH = RPW // CH
NBUF = 3
CFULL = (C // 16) * 16
CTAIL = C - CFULL


def _sc_body(idx_hbm, out_hbm, idx_v, bufs, sem):
    wid = lax.axis_index("s") * 2 + lax.axis_index("c")
    base_row = wid * RPW
    pltpu.sync_copy(idx_hbm.at[pl.ds(base_row, RPW)], idx_v)

    lanes = lax.iota(jnp.int32, 16)
    zeros = jnp.zeros((16,), jnp.float32)
    ones = jnp.ones((16,), jnp.float32)
    tail_mask = lanes < CTAIL

    for buf in bufs:
        def _zero_row(r, _, buf=buf):
            for cc in range(CFULL // 16):
                buf[r, pl.ds(cc * 16, 16)] = zeros
            rows = jnp.full((16,), r, jnp.int32)
            plsc.store_scatter(buf, [rows, CFULL + lanes], zeros,
                               mask=tail_mask)
            return _
        lax.fori_loop(0, CH, _zero_row, 0)

    copies = [None] * NCH
    for c in range(NCH):
        b = bufs[c % NBUF]
        if c >= NBUF:
            copies[c - NBUF].wait()
            for g in range(CH // 16):
                oldcols = idx_v[pl.ds((c - NBUF) * CH + g * 16, 16)]
                plsc.store_scatter(b, [lanes + g * 16, oldcols], zeros)
        for g in range(CH // 16):
            cols = idx_v[pl.ds(c * CH + g * 16, 16)]
            plsc.store_scatter(b, [lanes + g * 16, cols], ones)
        copies[c] = pltpu.async_copy(
            b, out_hbm.at[pl.ds(base_row + c * CH, CH)], sem.at[c % NBUF])
    for c in range(NCH - NBUF, NCH):
        copies[c].wait()


def kernel(inputs):
    idx = inputs.astype(jnp.int32)
    mesh = plsc.VectorSubcoreMesh(core_axis_name="c", subcore_axis_name="s")
    run = functools.partial(
        pl.kernel,
        mesh=mesh,
        out_type=jax.ShapeDtypeStruct((N, C), jnp.float32),
        scratch_types=[
            pltpu.VMEM((RPW,), jnp.int32),
            tuple(pltpu.VMEM((CH, C), jnp.float32) for _ in range(NBUF)),
            pltpu.SemaphoreType.DMA((NBUF,)),
        ],
        compiler_params=pltpu.CompilerParams(needs_layout_passes=False),
    )(_sc_body)
    return run(idx)
